# baseline (speedup 1.0000x reference)
import functools

import jax
import jax.numpy as jnp
from jax import lax
from jax.experimental import pallas as pl
from jax.experimental.pallas import tpu as pltpu
from jax.experimental.pallas import tpu_sc as plsc

_F32 = jnp.float32

NN = 10000
NP = 10240
EE = 160000
ER = 1280
EP = ER * 128
ROWS_T = ER // 16
ER64 = EP // 64
ROWS64_T = ER64 // 16
NPT = NP // 16
BN = 512

_MESH = plsc.VectorSubcoreMesh(core_axis_name="c", subcore_axis_name="s")


@functools.partial(
    pl.kernel,
    out_type=jax.ShapeDtypeStruct((2 * NP, 128), _F32),
    mesh=_MESH,
    scratch_types=[
        pltpu.VMEM((ROWS_T // 2, 128), jnp.int32),
        pltpu.VMEM((128, 128), _F32),
        pltpu.VMEM_SHARED((NP, 128), _F32),
    ],
)
def _deg_kernel(dst_hbm, deg_hbm, dst_v, rows_v, deg_sh):
    c = lax.axis_index("c")
    w = lax.axis_index("s")

    pltpu.sync_copy(dst_hbm.at[pl.ds(c * (ER // 2) + w * (ROWS_T // 2), ROWS_T // 2)], dst_v)

    def _fill(val):
        def body(j, carry):
            for m in range(8):
                rows_v[j, pl.ds(m * 16, 16)] = jnp.full((16,), val, _F32)
            return carry
        lax.fori_loop(0, 128, body, 0)

    _fill(0.0)
    for q in range(NPT // 128):
        pltpu.sync_copy(rows_v, deg_sh.at[pl.ds(w * NPT + q * 128, 128)])
    _fill(1.0)
    plsc.subcore_barrier()

    def edge(j, carry):
        pltpu.sync_copy(rows_v, deg_sh.at[dst_v.at[j]], add=True)
        return carry
    lax.fori_loop(0, ROWS_T // 2, edge, 0)

    plsc.subcore_barrier()
    pltpu.sync_copy(deg_sh.at[pl.ds(w * NPT, NPT)],
                    deg_hbm.at[pl.ds(c * NP + w * NPT, NPT)])


@functools.partial(
    pl.kernel,
    out_type=jax.ShapeDtypeStruct((4 * NP, 128), _F32),
    mesh=_MESH,
    scratch_types=[
        pltpu.VMEM((ROWS64_T // 4, 64), jnp.int32),
        pltpu.VMEM((ROWS64_T // 4, 64), jnp.int32),
        pltpu.VMEM((4, 64, 128), _F32),
        pltpu.VMEM_SHARED((NP, 128), _F32),
        pltpu.SemaphoreType.DMA,
        pltpu.SemaphoreType.DMA,
    ],
)
def _spmm_kernel(g_hbm, src_hbm, dst_hbm, s_hbm,
                 src_v, dst_v, rows_v, acc_sh, gsem, ssem):
    c = lax.axis_index("c")
    w = lax.axis_index("s")
    HR = ROWS64_T // 4

    def fire_gather(j, b):
        pltpu.async_copy(g_hbm.at[src_v.at[j]], rows_v.at[b], gsem)

    def drain_gather(j, b):
        pltpu.make_async_copy(g_hbm.at[src_v.at[j]], rows_v.at[b], gsem).wait()

    def fire_scatter(j, b):
        pltpu.async_copy(rows_v.at[b], acc_sh.at[dst_v.at[j]], ssem, add=True)

    def drain_scatter(j, b):
        pltpu.make_async_copy(rows_v.at[b], acc_sh.at[dst_v.at[j]], ssem).wait()

    for k in range(2):
        base = (c * 2 + k) * NP
        plsc.subcore_barrier()

        def zrow(j, carry):
            for m in range(8):
                rows_v[0, j, pl.ds(m * 16, 16)] = jnp.zeros((16,), _F32)
                rows_v[1, j, pl.ds(m * 16, 16)] = jnp.zeros((16,), _F32)
            return carry
        lax.fori_loop(0, 64, zrow, 0)
        for q in range(NPT // 128):
            pltpu.sync_copy(rows_v.at[0], acc_sh.at[pl.ds(w * NPT + q * 128, 64)])
            pltpu.sync_copy(rows_v.at[1], acc_sh.at[pl.ds(w * NPT + q * 128 + 64, 64)])
        plsc.subcore_barrier()

        for hh in range(4):
            hbase = w * ROWS64_T + hh * HR
            pltpu.sync_copy(src_hbm.at[pl.ds(hbase, HR)], src_v)
            pltpu.sync_copy(dst_hbm.at[pl.ds(hbase, HR)], dst_v)

            def off(j, carry):
                for m in range(4):
                    sl = pl.ds(m * 16, 16)
                    src_v[j, sl] = src_v[j, sl] + base
                return carry
            lax.fori_loop(0, HR, off, 0)

            fire_gather(0, 0)
            fire_gather(1, 1)
            def step(r4, carry):
                for pp in range(4):
                    r = 4 * r4 + pp
                    @pl.when(r > 1)
                    def _():
                        drain_scatter(r - 2, (pp + 2) % 4)
                    @pl.when(r < HR - 2)
                    def _():
                        fire_gather(r + 2, (pp + 2) % 4)
                    drain_gather(r, pp)
                    fire_scatter(r, pp)
                return carry
            lax.fori_loop(0, HR // 4, step, 0)
            drain_scatter(HR - 2, 2)
            drain_scatter(HR - 1, 3)

        plsc.subcore_barrier()
        pltpu.sync_copy(acc_sh.at[pl.ds(w * NPT, NPT)],
                        s_hbm.at[pl.ds(base + w * NPT, NPT)])


def _tc1_body(x_ref, deg_ref, ahW_ref, ahb_ref, W0_ref, g_ref, dinv_ref):
    h0 = jnp.maximum(
        jnp.dot(x_ref[...], ahW_ref[...], preferred_element_type=_F32) + ahb_ref[...], 0.0)
    deg = deg_ref[0][:, :1] + deg_ref[1][:, :1] + 1.0
    dinv = lax.rsqrt(deg)
    t = jnp.dot(h0, W0_ref[...], preferred_element_type=_F32) * dinv
    for k in range(4):
        g_ref[k] = t[:, k * 128:(k + 1) * 128]
    dinv_ref[...] = jnp.broadcast_to(dinv, (BN, 128))


def _tc_mid_body(s_ref, gp_ref, dinv_ref, b_ref, W_ref, h_ref, g_ref):
    dinv = dinv_ref[...][:, :1]
    m = jnp.concatenate([s_ref[k] + gp_ref[k] for k in range(4)], axis=1)
    h = jnp.maximum(dinv * m + b_ref[...], 0.0)
    h_ref[...] = h
    t = jnp.dot(h, W_ref[...], preferred_element_type=_F32) * dinv
    for k in range(4):
        g_ref[k] = t[:, k * 128:(k + 1) * 128]


def _tc4_body(s_ref, g2_ref, dinv_ref, b2_ref, h1_ref, h2_ref, x_ref,
              oW1_ref, ob1_ref, oW2_ref, ob2_ref, pW_ref, pb_ref,
              cW1_ref, cb1_ref, cW2_ref, cb2_ref, out_ref):
    dinv = dinv_ref[...][:, :1]
    m = jnp.concatenate([s_ref[k] + g2_ref[k] for k in range(4)], axis=1)
    h3 = jnp.maximum(dinv * m + b2_ref[...], 0.0)
    hs = (h1_ref[...], h2_ref[...], h3)
    acc = jnp.broadcast_to(cb1_ref[...], (BN, 768))
    for l in range(3):
        e = jnp.maximum(
            jnp.dot(hs[l], oW1_ref[...], preferred_element_type=_F32) + ob1_ref[...], 0.0)
        e = jnp.dot(e, oW2_ref[...], preferred_element_type=_F32) + ob2_ref[...]
        acc = acc + jnp.dot(e, cW1_ref[l], preferred_element_type=_F32)
    y = jax.nn.sigmoid(
        jnp.dot(x_ref[...], pW_ref[...], preferred_element_type=_F32) + pb_ref[...])
    acc = acc + jnp.dot(y, cW1_ref[3], preferred_element_type=_F32)
    f = jnp.maximum(acc, 0.0)
    out_ref[...] = jnp.dot(f, cW2_ref[...], preferred_element_type=_F32) + cb2_ref[...]


def _full(shape):
    return pl.BlockSpec(shape, lambda i: tuple(0 for _ in shape))


def _row(width):
    return pl.BlockSpec((BN, width), lambda i: (i, 0))


def _tc1(xp, deg2, ahW, ahb, W0):
    return pl.pallas_call(
        _tc1_body,
        grid=(NP // BN,),
        in_specs=[
            _row(768),
            pl.BlockSpec((2, BN, 128), lambda i: (0, i, 0)),
            _full((768, 768)),
            _full((1, 768)),
            _full((768, 512)),
        ],
        out_specs=[
            pl.BlockSpec((4, BN, 128), lambda i: (0, i, 0)),
            _row(128),
        ],
        out_shape=[
            jax.ShapeDtypeStruct((4, NP, 128), _F32),
            jax.ShapeDtypeStruct((NP, 128), _F32),
        ],
    )(xp, deg2, ahW, ahb, W0)


def _tc_mid(s, gp, dinv, b, W):
    return pl.pallas_call(
        _tc_mid_body,
        grid=(NP // BN,),
        in_specs=[
            pl.BlockSpec((4, BN, 128), lambda i: (0, i, 0)),
            pl.BlockSpec((4, BN, 128), lambda i: (0, i, 0)),
            _row(128),
            _full((1, 512)),
            _full((512, 512)),
        ],
        out_specs=[
            _row(512),
            pl.BlockSpec((4, BN, 128), lambda i: (0, i, 0)),
        ],
        out_shape=[
            jax.ShapeDtypeStruct((NP, 512), _F32),
            jax.ShapeDtypeStruct((4, NP, 128), _F32),
        ],
    )(s, gp, dinv, b, W)


def _tc4(s, g, dinv, b2, h1, h2, xp, oW1, ob1, oW2, ob2, pW, pb, cW1, cb1, cW2, cb2):
    return pl.pallas_call(
        _tc4_body,
        grid=(NP // BN,),
        in_specs=[
            pl.BlockSpec((4, BN, 128), lambda i: (0, i, 0)),
            pl.BlockSpec((4, BN, 128), lambda i: (0, i, 0)),
            _row(128),
            _full((1, 512)),
            _row(512),
            _row(512),
            _row(768),
            _full((512, 512)),
            _full((1, 512)),
            _full((512, 768)),
            _full((1, 768)),
            _full((768, 768)),
            _full((1, 768)),
            _full((4, 768, 768)),
            _full((1, 768)),
            _full((768, 128)),
            _full((1, 128)),
        ],
        out_specs=_row(128),
        out_shape=jax.ShapeDtypeStruct((NP, 128), _F32),
    )(s, g, dinv, b2, h1, h2, xp, oW1, ob1, oW2, ob2, pW, pb, cW1, cb1, cW2, cb2)


def kernel(x, edge_index, batch, train_node_mask, edge_attr, ah_W, ah_b,
           W0, b0, W1, b1, W2, b2, oW1, ob1, oW2, ob2, pW, pb,
           cW1, cb1, cW2, cb2):
    src = edge_index[0]
    dst = edge_index[1]
    pad_e = EP - EE
    src2 = jnp.concatenate([src, jnp.zeros((pad_e,), jnp.int32)]).reshape(ER64, 64)
    dst2 = jnp.concatenate([dst, jnp.full((pad_e,), NP - 1, jnp.int32)]).reshape(ER64, 64)
    dstd = dst2.reshape(ER, 128)
    xp = jnp.pad(x, ((0, NP - NN), (0, 0)))

    deg2 = _deg_kernel(dstd).reshape(2, NP, 128)
    g0, dinv = _tc1(xp, deg2, ah_W, ah_b.reshape(1, -1), W0)

    s0 = _spmm_kernel(g0.reshape(4 * NP, 128), src2, dst2).reshape(4, NP, 128)
    h1, g1 = _tc_mid(s0, g0, dinv, b0.reshape(1, -1), W1)

    s1 = _spmm_kernel(g1.reshape(4 * NP, 128), src2, dst2).reshape(4, NP, 128)
    h2, g2 = _tc_mid(s1, g1, dinv, b1.reshape(1, -1), W2)

    s2 = _spmm_kernel(g2.reshape(4 * NP, 128), src2, dst2).reshape(4, NP, 128)

    cW2p = jnp.pad(cW2, ((0, 0), (0, 128 - 16)))
    cb2p = jnp.pad(cb2, (0, 128 - 16)).reshape(1, 128)
    pred = _tc4(s2, g2, dinv, b2.reshape(1, -1), h1, h2, xp,
                oW1, ob1.reshape(1, -1), oW2, ob2.reshape(1, -1),
                pW, pb.reshape(1, -1), cW1.reshape(4, 768, 768),
                cb1.reshape(1, -1), cW2p, cb2p)
    return pred[:NN, :16]


# speedup vs baseline: 1.1048x; 1.1048x over previous
import functools

import jax
import jax.numpy as jnp
from jax import lax
from jax.experimental import pallas as pl
from jax.experimental.pallas import tpu as pltpu
from jax.experimental.pallas import tpu_sc as plsc

_F32 = jnp.float32

NN = 10000
NP = 10240
EE = 160000
ER = 1280
EP = ER * 128
ROWS_T = ER // 16
NPT = NP // 16
BN = 1024

_MESH = plsc.VectorSubcoreMesh(core_axis_name="c", subcore_axis_name="s")


@functools.partial(
    pl.kernel,
    out_type=jax.ShapeDtypeStruct((2 * NP, 128), _F32),
    mesh=_MESH,
    scratch_types=[
        pltpu.VMEM((ROWS_T // 2, 128), jnp.int32),
        pltpu.VMEM((128, 128), _F32),
        pltpu.VMEM_SHARED((NP, 128), _F32),
    ],
)
def _deg_kernel(dst_hbm, deg_hbm, dst_v, rows_v, deg_sh):
    c = lax.axis_index("c")
    w = lax.axis_index("s")

    pltpu.sync_copy(dst_hbm.at[pl.ds(c * (ER // 2) + w * (ROWS_T // 2), ROWS_T // 2)], dst_v)

    def _fill(val):
        def body(j, carry):
            for m in range(8):
                rows_v[j, pl.ds(m * 16, 16)] = jnp.full((16,), val, _F32)
            return carry
        lax.fori_loop(0, 128, body, 0)

    _fill(0.0)
    for q in range(NPT // 128):
        pltpu.sync_copy(rows_v, deg_sh.at[pl.ds(w * NPT + q * 128, 128)])
    _fill(1.0)
    plsc.subcore_barrier()

    def edge(j, carry):
        pltpu.sync_copy(rows_v, deg_sh.at[dst_v.at[j]], add=True)
        return carry
    lax.fori_loop(0, ROWS_T // 2, edge, 0)

    plsc.subcore_barrier()
    pltpu.sync_copy(deg_sh.at[pl.ds(w * NPT, NPT)],
                    deg_hbm.at[pl.ds(c * NP + w * NPT, NPT)])


@functools.partial(
    pl.kernel,
    out_type=jax.ShapeDtypeStruct((4 * NP, 128), _F32),
    mesh=_MESH,
    scratch_types=[
        pltpu.VMEM((ROWS_T // 2, 128), jnp.int32),
        pltpu.VMEM((ROWS_T // 2, 128), jnp.int32),
        pltpu.VMEM((2, 128, 128), _F32),
        pltpu.VMEM_SHARED((NP, 128), _F32),
        pltpu.SemaphoreType.DMA,
        pltpu.SemaphoreType.DMA,
    ],
)
def _spmm_kernel(g_hbm, src_hbm, dst_hbm, s_hbm,
                 src_v, dst_v, rows_v, acc_sh, gsem, ssem):
    c = lax.axis_index("c")
    w = lax.axis_index("s")
    HR = ROWS_T // 2

    def fire_gather(j, b):
        pltpu.async_copy(g_hbm.at[src_v.at[j]], rows_v.at[b], gsem)

    def drain_gather(j, b):
        pltpu.make_async_copy(g_hbm.at[src_v.at[j]], rows_v.at[b], gsem).wait()

    def fire_scatter(j, b):
        pltpu.async_copy(rows_v.at[b], acc_sh.at[dst_v.at[j]], ssem, add=True)

    def drain_scatter(j, b):
        pltpu.make_async_copy(rows_v.at[b], acc_sh.at[dst_v.at[j]], ssem).wait()

    for k in range(2):
        base = (c * 2 + k) * NP
        plsc.subcore_barrier()

        def zrow(j, carry):
            for m in range(8):
                rows_v[0, j, pl.ds(m * 16, 16)] = jnp.zeros((16,), _F32)
            return carry
        lax.fori_loop(0, 128, zrow, 0)
        for q in range(NPT // 128):
            pltpu.sync_copy(rows_v.at[0], acc_sh.at[pl.ds(w * NPT + q * 128, 128)])
        plsc.subcore_barrier()

        for hh in range(2):
            hbase = w * ROWS_T + hh * HR
            pltpu.sync_copy(src_hbm.at[pl.ds(hbase, HR)], src_v)
            pltpu.sync_copy(dst_hbm.at[pl.ds(hbase, HR)], dst_v)

            def off(j, carry):
                for m in range(8):
                    sl = pl.ds(m * 16, 16)
                    src_v[j, sl] = src_v[j, sl] + base
                return carry
            lax.fori_loop(0, HR, off, 0)

            fire_gather(0, 0)
            def step(r2, carry):
                for pp in range(2):
                    r = 2 * r2 + pp
                    @pl.when(r > 0)
                    def _():
                        drain_scatter(r - 1, 1 - pp)
                    @pl.when(r < HR - 1)
                    def _():
                        fire_gather(r + 1, 1 - pp)
                    drain_gather(r, pp)
                    fire_scatter(r, pp)
                return carry
            lax.fori_loop(0, HR // 2, step, 0)
            drain_scatter(HR - 1, 1)

        plsc.subcore_barrier()
        pltpu.sync_copy(acc_sh.at[pl.ds(w * NPT, NPT)],
                        s_hbm.at[pl.ds(base + w * NPT, NPT)])


def _tc1_body(x_ref, deg_ref, ahW_ref, ahb_ref, W0_ref, g_ref, dinv_ref):
    h0 = jnp.maximum(
        jnp.dot(x_ref[...], ahW_ref[...], preferred_element_type=_F32) + ahb_ref[...], 0.0)
    deg = deg_ref[0][:, :1] + deg_ref[1][:, :1] + 1.0
    dinv = lax.rsqrt(deg)
    t = jnp.dot(h0.astype(jnp.bfloat16), W0_ref[...], preferred_element_type=_F32) * dinv
    for k in range(4):
        g_ref[k] = t[:, k * 128:(k + 1) * 128]
    dinv_ref[...] = jnp.broadcast_to(dinv, (BN, 128))


def _tc_mid_body(s_ref, gp_ref, dinv_ref, b_ref, W_ref, h_ref, g_ref):
    dinv = dinv_ref[...][:, :1]
    m = jnp.concatenate([s_ref[k] + gp_ref[k] for k in range(4)], axis=1)
    h = jnp.maximum(dinv * m + b_ref[...], 0.0)
    h_ref[...] = h
    t = jnp.dot(h.astype(jnp.bfloat16), W_ref[...], preferred_element_type=_F32) * dinv
    for k in range(4):
        g_ref[k] = t[:, k * 128:(k + 1) * 128]


def _tc4_body(s_ref, g2_ref, dinv_ref, b2_ref, h1_ref, h2_ref, x_ref,
              oW1_ref, ob1_ref, oW2_ref, ob2_ref, pW_ref, pb_ref,
              cW1_ref, cb1_ref, cW2_ref, cb2_ref, out_ref):
    bf = jnp.bfloat16
    dinv = dinv_ref[...][:, :1]
    m = jnp.concatenate([s_ref[k] + g2_ref[k] for k in range(4)], axis=1)
    h3 = jnp.maximum(dinv * m + b2_ref[...], 0.0)
    hs = (h1_ref[...], h2_ref[...], h3)
    acc = jnp.broadcast_to(cb1_ref[...], (BN, 768))
    for l in range(3):
        e = jnp.maximum(
            jnp.dot(hs[l].astype(bf), oW1_ref[...],
                    preferred_element_type=_F32) + ob1_ref[...], 0.0)
        e = jnp.dot(e.astype(bf), oW2_ref[...],
                    preferred_element_type=_F32) + ob2_ref[...]
        acc = acc + jnp.dot(e.astype(bf), cW1_ref[l], preferred_element_type=_F32)
    y = jax.nn.sigmoid(
        jnp.dot(x_ref[...], pW_ref[...], preferred_element_type=_F32) + pb_ref[...])
    acc = acc + jnp.dot(y.astype(bf), cW1_ref[3], preferred_element_type=_F32)
    f = jnp.maximum(acc, 0.0)
    out_ref[...] = jnp.dot(f.astype(bf), cW2_ref[...],
                           preferred_element_type=_F32) + cb2_ref[...]


def _full(shape):
    return pl.BlockSpec(shape, lambda i: tuple(0 for _ in shape))


def _row(width):
    return pl.BlockSpec((BN, width), lambda i: (i, 0))


def _tc1(xp, deg2, ahW, ahb, W0):
    return pl.pallas_call(
        _tc1_body,
        grid=(NP // BN,),
        in_specs=[
            _row(768),
            pl.BlockSpec((2, BN, 128), lambda i: (0, i, 0)),
            _full((768, 768)),
            _full((1, 768)),
            _full((768, 512)),
        ],
        out_specs=[
            pl.BlockSpec((4, BN, 128), lambda i: (0, i, 0)),
            _row(128),
        ],
        out_shape=[
            jax.ShapeDtypeStruct((4, NP, 128), _F32),
            jax.ShapeDtypeStruct((NP, 128), _F32),
        ],
    )(xp, deg2, ahW, ahb, W0)


def _tc_mid(s, gp, dinv, b, W):
    return pl.pallas_call(
        _tc_mid_body,
        grid=(NP // BN,),
        in_specs=[
            pl.BlockSpec((4, BN, 128), lambda i: (0, i, 0)),
            pl.BlockSpec((4, BN, 128), lambda i: (0, i, 0)),
            _row(128),
            _full((1, 512)),
            _full((512, 512)),
        ],
        out_specs=[
            _row(512),
            pl.BlockSpec((4, BN, 128), lambda i: (0, i, 0)),
        ],
        out_shape=[
            jax.ShapeDtypeStruct((NP, 512), _F32),
            jax.ShapeDtypeStruct((4, NP, 128), _F32),
        ],
    )(s, gp, dinv, b, W)


def _tc4(s, g, dinv, b2, h1, h2, xp, oW1, ob1, oW2, ob2, pW, pb, cW1, cb1, cW2, cb2):
    return pl.pallas_call(
        _tc4_body,
        grid=(NP // BN,),
        in_specs=[
            pl.BlockSpec((4, BN, 128), lambda i: (0, i, 0)),
            pl.BlockSpec((4, BN, 128), lambda i: (0, i, 0)),
            _row(128),
            _full((1, 512)),
            _row(512),
            _row(512),
            _row(768),
            _full((512, 512)),
            _full((1, 512)),
            _full((512, 768)),
            _full((1, 768)),
            _full((768, 768)),
            _full((1, 768)),
            _full((4, 768, 768)),
            _full((1, 768)),
            _full((768, 128)),
            _full((1, 128)),
        ],
        out_specs=_row(128),
        out_shape=jax.ShapeDtypeStruct((NP, 128), _F32),
    )(s, g, dinv, b2, h1, h2, xp, oW1, ob1, oW2, ob2, pW, pb, cW1, cb1, cW2, cb2)


def kernel(x, edge_index, batch, train_node_mask, edge_attr, ah_W, ah_b,
           W0, b0, W1, b1, W2, b2, oW1, ob1, oW2, ob2, pW, pb,
           cW1, cb1, cW2, cb2):
    src = edge_index[0]
    dst = edge_index[1]
    pad_e = EP - EE
    src2 = jnp.concatenate([src, jnp.zeros((pad_e,), jnp.int32)]).reshape(ER, 128)
    dst2 = jnp.concatenate([dst, jnp.full((pad_e,), NP - 1, jnp.int32)]).reshape(ER, 128)
    xp = jnp.pad(x, ((0, NP - NN), (0, 0)))

    bf = jnp.bfloat16
    xpb = xp.astype(bf)
    deg2 = _deg_kernel(dst2).reshape(2, NP, 128)
    g0, dinv = _tc1(xpb, deg2, ah_W.astype(bf), ah_b.reshape(1, -1), W0.astype(bf))

    s0 = _spmm_kernel(g0.reshape(4 * NP, 128), src2, dst2).reshape(4, NP, 128)
    h1, g1 = _tc_mid(s0, g0, dinv, b0.reshape(1, -1), W1.astype(bf))

    s1 = _spmm_kernel(g1.reshape(4 * NP, 128), src2, dst2).reshape(4, NP, 128)
    h2, g2 = _tc_mid(s1, g1, dinv, b1.reshape(1, -1), W2.astype(bf))

    s2 = _spmm_kernel(g2.reshape(4 * NP, 128), src2, dst2).reshape(4, NP, 128)

    cW2p = jnp.pad(cW2, ((0, 0), (0, 128 - 16))).astype(bf)
    cb2p = jnp.pad(cb2, (0, 128 - 16)).reshape(1, 128)
    pred = _tc4(s2, g2, dinv, b2.reshape(1, -1), h1, h2, xpb,
                oW1.astype(bf), ob1.reshape(1, -1), oW2.astype(bf), ob2.reshape(1, -1),
                pW.astype(bf), pb.reshape(1, -1), cW1.reshape(4, 768, 768).astype(bf),
                cb1.reshape(1, -1), cW2p, cb2p)
    return pred[:NN, :16]
